# baseline (speedup 1.0000x reference)
import functools

import jax
import jax.numpy as jnp
import numpy as np
from jax import lax
from jax.experimental import pallas as pl
from jax.experimental.pallas import tpu as pltpu
from jax.experimental.pallas import tpu_sc as plsc

NC, NS, L = 2, 16, 16
NW = NC * NS
N1 = 16384
D = 256

ROWS_PER_W = N1 // NW
CH = 4
N_ITERS = ROWS_PER_W // CH

R = 1024
G = R // 16
STEPS = N1 // R

_S_SEL = jnp.asarray(np.repeat(np.eye(G, dtype=np.float32), 16, axis=1) / 16.0)


def _lrelu(x):
    return jnp.where(x > 0, x, 0.01 * x)


def _sc_body(x2_hbm, a1_hbm, in_bufs, out_bufs, sems_in, sems_out):
    wid = lax.axis_index("s") * NC + lax.axis_index("c")
    base_a1 = wid * ROWS_PER_W

    def in_copy(b, ci):
        return pltpu.make_async_copy(
            x2_hbm.at[pl.ds((base_a1 + ci * CH) * 16, CH * 16)],
            in_bufs[b], sems_in[b])

    def out_copy(b, ci):
        return pltpu.make_async_copy(
            out_bufs[b], a1_hbm.at[pl.ds(base_a1 + ci * CH, CH)],
            sems_out[b])

    in_copy(0, 0).start()
    in_copy(1, 1).start()

    def step(k, _):
        for b in range(2):
            ci = 2 * k + b
            in_copy(b, ci).wait()

            @pl.when(k > 0)
            def _drain():
                out_copy(b, ci - 2).wait()

            ibuf = in_bufs[b]
            obuf = out_bufs[b]

            for g in range(CH):
                for j in range(D // L):
                    vs = [ibuf[g * 16 + r, pl.ds(j * L, L)]
                          for r in range(16)]
                    while len(vs) > 1:
                        vs = [vs[t] + vs[t + 1] for t in range(0, len(vs), 2)]
                    obuf[g, pl.ds(j * L, L)] = vs[0] * (1.0 / 16.0)
            out_copy(b, ci).start()

            @pl.when(ci + 2 < N_ITERS)
            def _prefetch():
                in_copy(b, ci + 2).start()
        return _

    lax.fori_loop(0, N_ITERS // 2, step, None)
    out_copy(0, N_ITERS - 2).wait()
    out_copy(1, N_ITERS - 1).wait()


@functools.partial(
    pl.kernel,
    mesh=plsc.VectorSubcoreMesh(core_axis_name="c", subcore_axis_name="s"),
    out_type=jax.ShapeDtypeStruct((N1, D), jnp.float32),
    scratch_types=[
        pltpu.VMEM((CH * 16, D), jnp.float32),
        pltpu.VMEM((CH * 16, D), jnp.float32),
        pltpu.VMEM((CH, D), jnp.float32),
        pltpu.VMEM((CH, D), jnp.float32),
        pltpu.SemaphoreType.DMA,
        pltpu.SemaphoreType.DMA,
        pltpu.SemaphoreType.DMA,
        pltpu.SemaphoreType.DMA,
    ],
)
def _sc_mean16(x2_hbm, a1_hbm, in0, in1, out0, out1, si0, si1, so0, so1):
    _sc_body(x2_hbm, a1_hbm, (in0, in1), (out0, out1), (si0, si1), (so0, so1))


def _tc_kernel(a1_ref, x1_ref, x0_ref, S_ref, Wn0_ref, Ws0_ref, Wn1_ref,
               Ws1_ref, out_ref, b_acc, a0_acc):
    i = pl.program_id(0)
    Wn0 = Wn0_ref[...]
    Ws0 = Ws0_ref[...]
    S = S_ref[...]

    a1 = a1_ref[...]
    x1b = x1_ref[...]
    h1 = _lrelu(
        jnp.dot(x1b, Ws0, preferred_element_type=jnp.float32)
        + jnp.dot(a1, Wn0, preferred_element_type=jnp.float32))
    b_acc[pl.ds(i * G, G), :] = jnp.dot(S, h1,
                                        preferred_element_type=jnp.float32)
    a0_acc[pl.ds(i * G, G), :] = jnp.dot(S, x1b,
                                         preferred_element_type=jnp.float32)

    @pl.when(i == STEPS - 1)
    def _final():
        x0 = x0_ref[...]
        h0 = _lrelu(
            jnp.dot(x0, Ws0, preferred_element_type=jnp.float32)
            + jnp.dot(a0_acc[...], Wn0, preferred_element_type=jnp.float32))
        out_ref[...] = (
            jnp.dot(h0, Ws1_ref[...], preferred_element_type=jnp.float32)
            + jnp.dot(b_acc[...], Wn1_ref[...],
                      preferred_element_type=jnp.float32))


def kernel(x0, x1, x2, Wn0, Ws0, Wn1, Ws1):
    a1 = _sc_mean16(x2)
    return pl.pallas_call(
        _tc_kernel,
        grid=(STEPS,),
        in_specs=[
            pl.BlockSpec((R, 256), lambda i: (i, 0)),
            pl.BlockSpec((R, 256), lambda i: (i, 0)),
            pl.BlockSpec((1024, 256), lambda i: (0, 0)),
            pl.BlockSpec((G, R), lambda i: (0, 0)),
            pl.BlockSpec((256, 256), lambda i: (0, 0)),
            pl.BlockSpec((256, 256), lambda i: (0, 0)),
            pl.BlockSpec((256, 128), lambda i: (0, 0)),
            pl.BlockSpec((256, 128), lambda i: (0, 0)),
        ],
        out_specs=pl.BlockSpec((1024, 128), lambda i: (0, 0)),
        out_shape=jax.ShapeDtypeStruct((1024, 128), jnp.float32),
        scratch_shapes=[
            pltpu.VMEM((1024, 256), jnp.float32),
            pltpu.VMEM((1024, 256), jnp.float32),
        ],
    )(a1, x1, x0, _S_SEL, Wn0, Ws0, Wn1, Ws1)


# speedup vs baseline: 2.0978x; 2.0978x over previous
import functools

import jax
import jax.numpy as jnp
import numpy as np
from jax import lax
from jax.experimental import pallas as pl
from jax.experimental.pallas import tpu as pltpu
from jax.experimental.pallas import tpu_sc as plsc

NC, NS, L = 2, 16, 16
NW = NC * NS
N1 = 16384
D = 256

ROWS_PER_W = N1 // NW
CH = 8
N_ITERS = ROWS_PER_W // CH

R = 1024
G = R // 16
STEPS = N1 // R

_S_SEL = jnp.asarray(np.repeat(np.eye(G, dtype=np.float32), 16, axis=1) / 16.0)


def _lrelu(x):
    return jnp.where(x > 0, x, 0.01 * x)


def _sc_body(x2_hbm, a1_hbm, in_bufs, out_bufs, sems_in, sems_out):
    wid = lax.axis_index("s") * NC + lax.axis_index("c")
    base_a1 = wid * ROWS_PER_W

    def in_copy(b, ci):
        return pltpu.make_async_copy(
            x2_hbm.at[pl.ds((base_a1 + ci * CH) * 16, CH * 16)],
            in_bufs[b], sems_in[b])

    def out_copy(b, ci):
        return pltpu.make_async_copy(
            out_bufs[b], a1_hbm.at[pl.ds(base_a1 + ci * CH, CH)],
            sems_out[b])

    in_copy(0, 0).start()
    in_copy(1, 1).start()

    def step(k, _):
        for b in range(2):
            ci = 2 * k + b
            in_copy(b, ci).wait()

            @pl.when(k > 0)
            def _drain():
                out_copy(b, ci - 2).wait()

            ibuf = in_bufs[b]
            obuf = out_bufs[b]

            def row(g, _):
                for j in range(D // L):
                    vs = [ibuf[g * 16 + r, pl.ds(j * L, L)]
                          for r in range(16)]
                    while len(vs) > 1:
                        vs = [vs[t] + vs[t + 1] for t in range(0, len(vs), 2)]
                    obuf[g, pl.ds(j * L, L)] = vs[0] * (1.0 / 16.0)
                return _

            lax.fori_loop(0, CH, row, None)
            out_copy(b, ci).start()

            @pl.when(ci + 2 < N_ITERS)
            def _prefetch():
                in_copy(b, ci + 2).start()
        return _

    lax.fori_loop(0, N_ITERS // 2, step, None)
    out_copy(0, N_ITERS - 2).wait()
    out_copy(1, N_ITERS - 1).wait()


@functools.partial(
    pl.kernel,
    mesh=plsc.VectorSubcoreMesh(core_axis_name="c", subcore_axis_name="s"),
    out_type=jax.ShapeDtypeStruct((N1, D), jnp.float32),
    scratch_types=[
        pltpu.VMEM((CH * 16, D), jnp.float32),
        pltpu.VMEM((CH * 16, D), jnp.float32),
        pltpu.VMEM((CH, D), jnp.float32),
        pltpu.VMEM((CH, D), jnp.float32),
        pltpu.SemaphoreType.DMA,
        pltpu.SemaphoreType.DMA,
        pltpu.SemaphoreType.DMA,
        pltpu.SemaphoreType.DMA,
    ],
)
def _sc_mean16(x2_hbm, a1_hbm, in0, in1, out0, out1, si0, si1, so0, so1):
    _sc_body(x2_hbm, a1_hbm, (in0, in1), (out0, out1), (si0, si1), (so0, so1))


def _tc_kernel(a1_ref, x1_ref, x0_ref, S_ref, Wn0_ref, Ws0_ref, Wn1_ref,
               Ws1_ref, out_ref, b_acc, a0_acc):
    i = pl.program_id(0)
    Wn0 = Wn0_ref[...]
    Ws0 = Ws0_ref[...]
    S = S_ref[...]

    x2b = a1_ref[...]
    a1 = jnp.mean(x2b.reshape(R, 16, 256), axis=1)
    x1b = x1_ref[...]
    h1 = _lrelu(
        jnp.dot(x1b, Ws0, preferred_element_type=jnp.float32)
        + jnp.dot(a1, Wn0, preferred_element_type=jnp.float32))
    b_acc[pl.ds(i * G, G), :] = jnp.dot(S, h1,
                                        preferred_element_type=jnp.float32)
    a0_acc[pl.ds(i * G, G), :] = jnp.dot(S, x1b,
                                         preferred_element_type=jnp.float32)

    @pl.when(i == STEPS - 1)
    def _final():
        x0 = x0_ref[...]
        h0 = _lrelu(
            jnp.dot(x0, Ws0, preferred_element_type=jnp.float32)
            + jnp.dot(a0_acc[...], Wn0, preferred_element_type=jnp.float32))
        out_ref[...] = (
            jnp.dot(h0, Ws1_ref[...], preferred_element_type=jnp.float32)
            + jnp.dot(b_acc[...], Wn1_ref[...],
                      preferred_element_type=jnp.float32))


def kernel(x0, x1, x2, Wn0, Ws0, Wn1, Ws1):
    a1 = _sc_mean16(x2)
    out = pl.pallas_call(
        _tc_kernel,
        grid=(STEPS,),
        in_specs=[
            pl.BlockSpec((R * 16, 256), lambda i: (i, 0)),
            pl.BlockSpec((R, 256), lambda i: (i, 0)),
            pl.BlockSpec((1024, 256), lambda i: (0, 0)),
            pl.BlockSpec((G, R), lambda i: (0, 0)),
            pl.BlockSpec((256, 256), lambda i: (0, 0)),
            pl.BlockSpec((256, 256), lambda i: (0, 0)),
            pl.BlockSpec((256, 128), lambda i: (0, 0)),
            pl.BlockSpec((256, 128), lambda i: (0, 0)),
        ],
        out_specs=pl.BlockSpec((1024, 128), lambda i: (0, 0)),
        out_shape=jax.ShapeDtypeStruct((1024, 128), jnp.float32),
        scratch_shapes=[
            pltpu.VMEM((1024, 256), jnp.float32),
            pltpu.VMEM((1024, 256), jnp.float32),
        ],
    )(x2, x1, x0, _S_SEL, Wn0, Ws0, Wn1, Ws1)
    return out + a1[:1024, :128] * 1e-30


# speedup vs baseline: 4.7206x; 2.2503x over previous
import jax
import jax.numpy as jnp
import numpy as np
from jax.experimental import pallas as pl
from jax.experimental.pallas import tpu as pltpu

R = 1024
G = R // 16
N1 = 16384
STEPS = N1 // R

_S_SEL = jnp.asarray(np.repeat(np.eye(G, dtype=np.float32), 16, axis=1) / 16.0)


def _lrelu(x):
    return jnp.where(x > 0, x, 0.01 * x)


def _sage_kernel(x2_ref, x1_ref, x0_ref, S_ref, Wn0_ref, Ws0_ref, Wn1_ref,
                 Ws1_ref, out_ref, b_acc, a0_acc):
    i = pl.program_id(0)
    Wn0 = Wn0_ref[...]
    Ws0 = Ws0_ref[...]
    S = S_ref[...]

    x2b = x2_ref[...]
    a1 = jnp.mean(x2b.reshape(R, 16, 256), axis=1)
    x1b = x1_ref[...]
    h1 = _lrelu(
        jnp.dot(x1b, Ws0, preferred_element_type=jnp.float32)
        + jnp.dot(a1, Wn0, preferred_element_type=jnp.float32))
    b_acc[pl.ds(i * G, G), :] = jnp.dot(S, h1,
                                        preferred_element_type=jnp.float32)
    a0_acc[pl.ds(i * G, G), :] = jnp.dot(S, x1b,
                                         preferred_element_type=jnp.float32)

    @pl.when(i == STEPS - 1)
    def _final():
        x0 = x0_ref[...]
        h0 = _lrelu(
            jnp.dot(x0, Ws0, preferred_element_type=jnp.float32)
            + jnp.dot(a0_acc[...], Wn0, preferred_element_type=jnp.float32))
        out_ref[...] = (
            jnp.dot(h0, Ws1_ref[...], preferred_element_type=jnp.float32)
            + jnp.dot(b_acc[...], Wn1_ref[...],
                      preferred_element_type=jnp.float32))


def kernel(x0, x1, x2, Wn0, Ws0, Wn1, Ws1):
    return pl.pallas_call(
        _sage_kernel,
        grid=(STEPS,),
        in_specs=[
            pl.BlockSpec((R * 16, 256), lambda i: (i, 0)),
            pl.BlockSpec((R, 256), lambda i: (i, 0)),
            pl.BlockSpec((1024, 256), lambda i: (0, 0)),
            pl.BlockSpec((G, R), lambda i: (0, 0)),
            pl.BlockSpec((256, 256), lambda i: (0, 0)),
            pl.BlockSpec((256, 256), lambda i: (0, 0)),
            pl.BlockSpec((256, 128), lambda i: (0, 0)),
            pl.BlockSpec((256, 128), lambda i: (0, 0)),
        ],
        out_specs=pl.BlockSpec((1024, 128), lambda i: (0, 0)),
        out_shape=jax.ShapeDtypeStruct((1024, 128), jnp.float32),
        scratch_shapes=[
            pltpu.VMEM((1024, 256), jnp.float32),
            pltpu.VMEM((1024, 256), jnp.float32),
        ],
    )(x2, x1, x0, _S_SEL, Wn0, Ws0, Wn1, Ws1)


# speedup vs baseline: 4.7217x; 1.0002x over previous
import jax
import jax.numpy as jnp
import numpy as np
from jax.experimental import pallas as pl
from jax.experimental.pallas import tpu as pltpu

R = 1024
G = R // 16
N1 = 16384
STEPS = N1 // R

_S_SEL = (np.repeat(np.eye(G, dtype=np.float32), 16, axis=1) / 16.0).astype(
    np.float32)


def _lrelu(x):
    return jnp.where(x > 0, x, 0.01 * x)


def _sage_kernel(x2_ref, x1_ref, x0_ref, S_ref, Wn0_ref, Ws0_ref, Wn1_ref,
                 Ws1_ref, out_ref, b_acc, a0_acc):
    i = pl.program_id(0)
    Wn0 = Wn0_ref[...]
    Ws0 = Ws0_ref[...]
    S = S_ref[...]

    x2b = x2_ref[...]
    a1 = jnp.mean(x2b.reshape(R, 16, 256), axis=1)
    x1b = x1_ref[...]
    h1 = _lrelu(
        jnp.dot(x1b, Ws0, preferred_element_type=jnp.float32)
        + jnp.dot(a1, Wn0, preferred_element_type=jnp.float32))
    b_acc[pl.ds(i * G, G), :] = jnp.dot(S, h1,
                                        preferred_element_type=jnp.float32)
    a0_acc[pl.ds(i * G, G), :] = jnp.dot(S, x1b,
                                         preferred_element_type=jnp.float32)

    @pl.when(i == STEPS - 1)
    def _final():
        x0 = x0_ref[...]
        h0 = _lrelu(
            jnp.dot(x0, Ws0, preferred_element_type=jnp.float32)
            + jnp.dot(a0_acc[...], Wn0, preferred_element_type=jnp.float32))
        out_ref[...] = (
            jnp.dot(h0, Ws1_ref[...], preferred_element_type=jnp.float32)
            + jnp.dot(b_acc[...], Wn1_ref[...],
                      preferred_element_type=jnp.float32))


def kernel(x0, x1, x2, Wn0, Ws0, Wn1, Ws1):
    return pl.pallas_call(
        _sage_kernel,
        grid=(STEPS,),
        in_specs=[
            pl.BlockSpec((R * 16, 256), lambda i: (i, 0)),
            pl.BlockSpec((R, 256), lambda i: (i, 0)),
            pl.BlockSpec((1024, 256), lambda i: (0, 0)),
            pl.BlockSpec((G, R), lambda i: (0, 0)),
            pl.BlockSpec((256, 256), lambda i: (0, 0)),
            pl.BlockSpec((256, 256), lambda i: (0, 0)),
            pl.BlockSpec((256, 128), lambda i: (0, 0)),
            pl.BlockSpec((256, 128), lambda i: (0, 0)),
        ],
        out_specs=pl.BlockSpec((1024, 128), lambda i: (0, 0)),
        out_shape=jax.ShapeDtypeStruct((1024, 128), jnp.float32),
        scratch_shapes=[
            pltpu.VMEM((1024, 256), jnp.float32),
            pltpu.VMEM((1024, 256), jnp.float32),
        ],
    )(x2, x1, x0, _S_SEL, Wn0, Ws0, Wn1, Ws1)
